# baseline (speedup 1.0000x reference)
import functools

import jax
import jax.numpy as jnp
from jax import lax
from jax.experimental import pallas as pl
from jax.experimental.pallas import tpu as pltpu
from jax.experimental.pallas import tpu_sc as plsc

_NC = 2
_NS = 16
_NW = _NC * _NS
_L = 16
_CH = 16


@functools.lru_cache(maxsize=None)
def _build_sc_call(B, N):
    NP = -(-N // 128) * 128
    NCH = (B // _NW) // _CH
    FS = N // _L
    REM = N % _L
    TOFF = N - _L

    mesh = plsc.VectorSubcoreMesh(core_axis_name="c", subcore_axis_name="s")

    def body(p_hbm, t_hbm, c_hbm, out_hbm,
             idx_v, pbuf0, pbuf1, gbuf0, gbuf1, obuf,
             lsem0, lsem1, gsem0, gsem1):
        cid = lax.axis_index("c")
        sid = lax.axis_index("s")
        wid = cid * _NS + sid
        zv = jnp.zeros((_L,), jnp.float32)
        lanes = lax.iota(jnp.int32, _L)
        tmask = lanes >= (_L - REM)

        pltpu.sync_copy(t_hbm.at[wid], idx_v)

        rowbase = wid * (NCH * _CH)
        pbufs = (pbuf0, pbuf1)
        gbufs = (gbuf0, gbuf1)
        lsems = (lsem0, lsem1)
        gsems = (gsem0, gsem1)

        def start(j):
            b = j % 2
            ldj = pltpu.async_copy(
                p_hbm.at[pl.ds(rowbase + j * _CH, _CH)], pbufs[b], lsems[b])
            gdj = pltpu.async_copy(
                c_hbm.at[idx_v.at[j]], gbufs[b], gsems[b])
            return ldj, gdj

        NA = 4

        def chunk_dot(pb, gb, acc):
            def fbody(jj, accs):
                off = jj * _L
                accs = list(accs)
                for r in range(_CH):
                    accs[r % NA] = (accs[r % NA]
                                    + pb[r, pl.ds(off, _L)]
                                    * gb[r, pl.ds(off, _L)])
                return tuple(accs)
            accs = lax.fori_loop(0, FS, fbody, (acc,) + (zv,) * (NA - 1))
            accs = list(accs)
            if REM:
                for r in range(_CH):
                    t = pb[r, pl.ds(TOFF, _L)] * gb[r, pl.ds(TOFF, _L)]
                    accs[r % NA] = accs[r % NA] + jnp.where(tmask, t, zv)
            out = accs[0]
            for a in accs[1:]:
                out = out + a
            return out

        acc = zv
        pend = [None, None]
        pend[0] = start(0)
        for j in range(NCH):
            b = j % 2
            ldj, gdj = pend[b]
            ldj.wait()
            gdj.wait()
            if j + 1 < NCH:
                pend[1 - b] = start(j + 1)
            acc = chunk_dot(pbufs[b], gbufs[b], acc)

        obuf[...] = acc
        pltpu.sync_copy(obuf, out_hbm.at[wid])

    return pl.kernel(
        body,
        out_type=jax.ShapeDtypeStruct((_NW, _L), jnp.float32),
        mesh=mesh,
        scratch_types=[
            pltpu.VMEM((NCH, _CH), jnp.int32),
            pltpu.VMEM((_CH, N), jnp.float32),
            pltpu.VMEM((_CH, N), jnp.float32),
            pltpu.VMEM((_CH, NP), jnp.float32),
            pltpu.VMEM((_CH, NP), jnp.float32),
            pltpu.VMEM((_L,), jnp.float32),
            pltpu.SemaphoreType.DMA,
            pltpu.SemaphoreType.DMA,
            pltpu.SemaphoreType.DMA,
            pltpu.SemaphoreType.DMA,
        ],
        name="otloss_sc",
    )


_R = 1024


@functools.lru_cache(maxsize=None)
def _build_tr_call(B, N, BSC, OFFT):
    def tr_body(pt_ref, o_ref):
        o_ref[...] = pt_ref[...].T

    return pl.pallas_call(
        tr_body,
        grid=(BSC // _R,),
        in_specs=[pl.BlockSpec((N, _R), lambda i: (0, OFFT + i))],
        out_specs=pl.BlockSpec((_R, N), lambda i: (i, 0)),
        out_shape=jax.ShapeDtypeStruct((BSC, N), jnp.float32),
        name="otloss_tr",
    )


@functools.lru_cache(maxsize=None)
def _build_tc_call(B, N, BSC):
    NB = (B - BSC) // _R
    OFF = BSC // _R

    def tc_body(t_ref, pt_ref, ct_ref, o_ref):
        t_row = t_ref[0]
        kio = lax.broadcasted_iota(jnp.int32, (N, _R), 0)
        gt = (kio == t_row).astype(jnp.bfloat16)
        dt = jnp.dot(ct_ref[...], gt, preferred_element_type=jnp.float32)
        s = jnp.sum(dt * pt_ref[...])

        @pl.when(pl.program_id(0) == 0)
        def _():
            o_ref[0, 0] = 0.0

        o_ref[0, 0] += s

    return pl.pallas_call(
        tc_body,
        grid=(NB,),
        in_specs=[
            pl.BlockSpec((1, 1, _R), lambda i: (i, 0, 0)),
            pl.BlockSpec((N, _R), lambda i: (0, OFF + i)),
            pl.BlockSpec((N, N), lambda i: (0, 0)),
        ],
        out_specs=pl.BlockSpec(memory_space=pltpu.SMEM),
        out_shape=jax.ShapeDtypeStruct((1, 1), jnp.float32),
        compiler_params=pltpu.CompilerParams(
            dimension_semantics=("arbitrary",)),
        name="otloss_tc",
    )


def kernel(output_probs, target_class, C):
    B, N = output_probs.shape
    NP = -(-N // 128) * 128
    BSC1 = 2 * (B // 16)
    BSC2 = 6 * (B // 16)
    BSC = BSC1 + BSC2
    tci = target_class.astype(jnp.int32)
    pt = output_probs.T
    c_pad = jnp.pad(C, ((0, 0), (0, NP - N)))

    idx1 = tci[:BSC1].reshape(_NW, (BSC1 // _NW) // _CH, _CH)
    p1 = _build_tr_call(B, N, BSC1, 0)(pt)
    partials1 = _build_sc_call(BSC1, N)(p1, idx1, c_pad)

    idx2 = tci[BSC1:BSC].reshape(_NW, (BSC2 // _NW) // _CH, _CH)
    p2 = _build_tr_call(B, N, BSC2, BSC1 // _R)(pt)
    partials2 = _build_sc_call(BSC2, N)(p2, idx2, c_pad)

    t3 = tci[BSC:].reshape((B - BSC) // _R, 1, _R)
    ct_b = C.T.astype(jnp.bfloat16)
    tc_sum = _build_tc_call(B, N, BSC)(t3, pt, ct_b)
    return (jnp.sum(partials1) + jnp.sum(partials2) + tc_sum[0, 0]) / B


# speedup vs baseline: 1.0894x; 1.0894x over previous
import functools

import jax
import jax.numpy as jnp
from jax import lax
from jax.experimental import pallas as pl
from jax.experimental.pallas import tpu as pltpu
from jax.experimental.pallas import tpu_sc as plsc

_NC = 2
_NS = 16
_NW = _NC * _NS
_L = 16
_CH = 16


@functools.lru_cache(maxsize=None)
def _build_sc_call(B, N):
    NP = -(-N // 128) * 128
    NCH = (B // _NW) // _CH
    FS = N // _L
    REM = N % _L
    TOFF = N - _L

    mesh = plsc.VectorSubcoreMesh(core_axis_name="c", subcore_axis_name="s")

    def body(p_hbm, t_hbm, c_hbm, out_hbm,
             idx_v, pbuf0, pbuf1, gbuf0, gbuf1, obuf,
             lsem0, lsem1, gsem0, gsem1):
        cid = lax.axis_index("c")
        sid = lax.axis_index("s")
        wid = cid * _NS + sid
        zv = jnp.zeros((_L,), jnp.float32)
        lanes = lax.iota(jnp.int32, _L)
        tmask = lanes >= (_L - REM)

        pltpu.sync_copy(t_hbm.at[wid], idx_v)

        rowbase = wid * (NCH * _CH)
        pbufs = (pbuf0, pbuf1)
        gbufs = (gbuf0, gbuf1)
        lsems = (lsem0, lsem1)
        gsems = (gsem0, gsem1)

        def start(j):
            b = j % 2
            ldj = pltpu.async_copy(
                p_hbm.at[pl.ds(rowbase + j * _CH, _CH)], pbufs[b], lsems[b])
            gdj = pltpu.async_copy(
                c_hbm.at[idx_v.at[j]], gbufs[b], gsems[b])
            return ldj, gdj

        NA = 4

        def chunk_dot(pb, gb, acc):
            def fbody(jj, accs):
                off = jj * _L
                accs = list(accs)
                for r in range(_CH):
                    accs[r % NA] = (accs[r % NA]
                                    + pb[r, pl.ds(off, _L)]
                                    * gb[r, pl.ds(off, _L)])
                return tuple(accs)
            accs = lax.fori_loop(0, FS, fbody, (acc,) + (zv,) * (NA - 1))
            accs = list(accs)
            if REM:
                for r in range(_CH):
                    t = pb[r, pl.ds(TOFF, _L)] * gb[r, pl.ds(TOFF, _L)]
                    accs[r % NA] = accs[r % NA] + jnp.where(tmask, t, zv)
            out = accs[0]
            for a in accs[1:]:
                out = out + a
            return out

        acc = zv
        pend = [None, None]
        pend[0] = start(0)
        for j in range(NCH):
            b = j % 2
            ldj, gdj = pend[b]
            ldj.wait()
            gdj.wait()
            if j + 1 < NCH:
                pend[1 - b] = start(j + 1)
            acc = chunk_dot(pbufs[b], gbufs[b], acc)

        obuf[...] = acc
        pltpu.sync_copy(obuf, out_hbm.at[wid])

    return pl.kernel(
        body,
        out_type=jax.ShapeDtypeStruct((_NW, _L), jnp.float32),
        mesh=mesh,
        scratch_types=[
            pltpu.VMEM((NCH, _CH), jnp.int32),
            pltpu.VMEM((_CH, N), jnp.float32),
            pltpu.VMEM((_CH, N), jnp.float32),
            pltpu.VMEM((_CH, NP), jnp.float32),
            pltpu.VMEM((_CH, NP), jnp.float32),
            pltpu.VMEM((_L,), jnp.float32),
            pltpu.SemaphoreType.DMA,
            pltpu.SemaphoreType.DMA,
            pltpu.SemaphoreType.DMA,
            pltpu.SemaphoreType.DMA,
        ],
        name="otloss_sc",
    )


_R = 1024


@functools.lru_cache(maxsize=None)
def _build_tr_call(B, N, BSC):
    def tr_body(pt_ref, o_ref):
        o_ref[...] = pt_ref[...].T

    return pl.pallas_call(
        tr_body,
        grid=(BSC // _R,),
        in_specs=[pl.BlockSpec((N, _R), lambda i: (0, i))],
        out_specs=pl.BlockSpec((_R, N), lambda i: (i, 0)),
        out_shape=jax.ShapeDtypeStruct((BSC, N), jnp.float32),
        name="otloss_tr",
    )


@functools.lru_cache(maxsize=None)
def _build_tc_call(B, N, BSC):
    NB = (B - BSC) // _R
    OFF = BSC // _R

    def tc_body(t_ref, pt_ref, ct_ref, o_ref):
        t_row = t_ref[0]
        kio = lax.broadcasted_iota(jnp.int32, (N, _R), 0)
        gt = (kio == t_row).astype(jnp.bfloat16)
        dt = jnp.dot(ct_ref[...], gt, preferred_element_type=jnp.float32)
        s = jnp.sum(dt * pt_ref[...])

        @pl.when(pl.program_id(0) == 0)
        def _():
            o_ref[0, 0] = 0.0

        o_ref[0, 0] += s

    return pl.pallas_call(
        tc_body,
        grid=(NB,),
        in_specs=[
            pl.BlockSpec((1, 1, _R), lambda i: (i, 0, 0)),
            pl.BlockSpec((N, _R), lambda i: (0, OFF + i)),
            pl.BlockSpec((N, N), lambda i: (0, 0)),
        ],
        out_specs=pl.BlockSpec(memory_space=pltpu.SMEM),
        out_shape=jax.ShapeDtypeStruct((1, 1), jnp.float32),
        compiler_params=pltpu.CompilerParams(
            dimension_semantics=("arbitrary",)),
        name="otloss_tc",
    )


def kernel(output_probs, target_class, C):
    B, N = output_probs.shape
    NP = -(-N // 128) * 128
    BSC = 6 * (B // 16)
    tci = target_class.astype(jnp.int32)
    idx3 = tci[:BSC].reshape(_NW, (BSC // _NW) // _CH, _CH)
    c_pad = jnp.pad(C, ((0, 0), (0, NP - N)))
    p_sc = _build_tr_call(B, N, BSC)(output_probs.T)
    partials = _build_sc_call(BSC, N)(p_sc, idx3, c_pad)
    t3 = tci[BSC:].reshape((B - BSC) // _R, 1, _R)
    ct_b = C.T.astype(jnp.bfloat16)
    tc_sum = _build_tc_call(B, N, BSC)(t3, output_probs.T, ct_b)
    return (jnp.sum(partials) + tc_sum[0, 0]) / B
